# Initial kernel scaffold; baseline (speedup 1.0000x reference)
import jax
import jax.numpy as jnp
from jax import lax
from jax.experimental import pallas as pl
from jax.experimental.pallas import tpu as pltpu
from jax.experimental.pallas import tpu_sc as plsc

N = 100000
E = 6400000
NWT = 1024
NT = 50000
NPAD = 102400

NC = 2
NS = 16
W = NC * NS

CH = 3200
R = CH // 128
NCHUNK = E // CH
CPW = -(-NCHUNK // W)

TROWS = 400
TPW = TROWS // NS


def _sc_body(x_hbm, w_hbm, u_hbm, v2d_hbm, wi_hbm, tgt_hbm,
             out0, out1, x_v, w_v, u_v, wi_v, idx2, val2, zbuf, acc, sem):
    c = lax.axis_index("c")
    s = lax.axis_index("s")
    wid = c * NS + s

    pltpu.sync_copy(x_hbm, x_v)
    pltpu.sync_copy(w_hbm, w_v)

    zv = jnp.zeros((16,), jnp.float32)
    for i in range(R * 8):
        val2[i // 8, pl.ds((i % 8) * 16, 16)] = zv
    for i in range(6400 // 16):
        zbuf[pl.ds(i * 16, 16)] = zv

    @pl.when(c == 0)
    def _():
        @pl.when(s < 15)
        def _():
            pltpu.sync_copy(x_hbm.at[pl.ds(s * 6400, 6400)],
                            acc.at[pl.ds(s * 6400, 6400)])

        @pl.when(s == 15)
        def _():
            pltpu.sync_copy(x_hbm.at[pl.ds(96000, 4000)],
                            acc.at[pl.ds(96000, 4000)])
            pltpu.sync_copy(zbuf.at[pl.ds(0, 2400)],
                            acc.at[pl.ds(100000, 2400)])

    @pl.when(c == 1)
    def _():
        pltpu.sync_copy(zbuf, acc.at[pl.ds(s * 6400, 6400)])

    plsc.subcore_barrier()

    pltpu.sync_copy(tgt_hbm.at[pl.ds(s * TPW, TPW), :], idx2)
    for j in range(TPW):
        pltpu.sync_copy(val2.at[j], acc.at[idx2.at[j]])

    plsc.subcore_barrier()

    def chunk_body(k, carry):
        cid = k * W + wid

        @pl.when(cid < NCHUNK)
        def _():
            base = cid * CH
            pltpu.sync_copy(u_hbm.at[pl.ds(base, CH)], u_v)
            pltpu.sync_copy(wi_hbm.at[pl.ds(base, CH)], wi_v)
            pltpu.sync_copy(v2d_hbm.at[pl.ds(cid * R, R), :], idx2)
            for i in range(CH // 16):
                ui = u_v[pl.ds(i * 16, 16)]
                wi = wi_v[pl.ds(i * 16, 16)]
                xg = plsc.load_gather(x_v, [ui])
                wg = plsc.load_gather(w_v, [wi])
                val2[i // 8, pl.ds((i % 8) * 16, 16)] = xg * wg
            descs = [pltpu.async_copy(val2.at[j], acc.at[idx2.at[j]], sem,
                                      add=True) for j in range(R)]
            for d in descs:
                d.wait()

        return carry

    lax.fori_loop(0, CPW, chunk_body, 0)

    plsc.subcore_barrier()

    @pl.when(c == 0)
    def _():
        pltpu.sync_copy(acc.at[pl.ds(s * 6400, 6400)],
                        out0.at[pl.ds(s * 6400, 6400)])

    @pl.when(c == 1)
    def _():
        pltpu.sync_copy(acc.at[pl.ds(s * 6400, 6400)],
                        out1.at[pl.ds(s * 6400, 6400)])


def _sc_scatter(xf, weights, u, v2d, w_idx, tgt2d):
    mesh = plsc.VectorSubcoreMesh(core_axis_name="c", subcore_axis_name="s",
                                  num_cores=NC, num_subcores=NS)
    return pl.kernel(
        _sc_body,
        out_type=(jax.ShapeDtypeStruct((NPAD,), jnp.float32),
                  jax.ShapeDtypeStruct((NPAD,), jnp.float32)),
        mesh=mesh,
        scratch_types=[
            pltpu.VMEM((N,), jnp.float32),
            pltpu.VMEM((NWT,), jnp.float32),
            pltpu.VMEM((CH,), jnp.int32),
            pltpu.VMEM((CH,), jnp.int32),
            pltpu.VMEM((R, 128), jnp.int32),
            pltpu.VMEM((R, 128), jnp.float32),
            pltpu.VMEM((6400,), jnp.float32),
            pltpu.VMEM_SHARED((NPAD,), jnp.float32),
            pltpu.SemaphoreType.DMA,
        ],
    )(xf, weights, u, v2d, w_idx, tgt2d)


def _add_body(a_ref, b_ref, o_ref):
    o_ref[...] = a_ref[...] + b_ref[...]


def kernel(x, weights, u, v, w_idx, targets):
    xf = x.reshape(N)
    v2d = v.reshape(E // 128, 128)
    pad = N + jnp.arange(TROWS * 128 - NT, dtype=jnp.int32)
    tgt2d = jnp.concatenate([targets, pad]).reshape(TROWS, 128)
    acc0, acc1 = _sc_scatter(xf, weights, u, v2d, w_idx, tgt2d)
    out2d = pl.pallas_call(
        _add_body,
        out_shape=jax.ShapeDtypeStruct((NPAD // 128, 128), jnp.float32),
    )(acc0.reshape(NPAD // 128, 128), acc1.reshape(NPAD // 128, 128))
    return out2d.reshape(NPAD)[:N].reshape(N, 1)


# speedup vs baseline: 313.9020x; 313.9020x over previous
import jax
import jax.numpy as jnp
from jax import lax
from jax.experimental import pallas as pl
from jax.experimental.pallas import tpu as pltpu
from jax.experimental.pallas import tpu_sc as plsc

N = 100000
E = 6400000
NWT = 1024
NT = 50000
NPAD = 102400

NC = 2
NS = 16
W = NC * NS

CH = 2048
R = CH // 128
NCHUNK = E // CH
CPW = -(-NCHUNK // W)

TROWS = 512
TPW = TROWS // NS


def _sc_body(x_hbm, w_hbm, u_hbm, v2d_hbm, wi_hbm, tgt_hbm,
             out0, out1, x_v, w_v, u_v, wi_v, idx2, val2, zbuf, acc, sem):
    c = lax.axis_index("c")
    s = lax.axis_index("s")
    wid = c * NS + s

    pltpu.sync_copy(x_hbm, x_v)
    pltpu.sync_copy(w_hbm, w_v)

    @pl.when(c == 0)
    def _():
        @pl.when(s < 15)
        def _():
            pltpu.sync_copy(x_hbm.at[pl.ds(s * 6400, 6400)], zbuf)
            pltpu.sync_copy(zbuf, acc.at[pl.ds(s * 6400, 6400)])

        @pl.when(s == 15)
        def _():
            pltpu.sync_copy(x_hbm.at[pl.ds(96000, 4000)], zbuf.at[pl.ds(0, 4000)])
            pltpu.sync_copy(zbuf.at[pl.ds(0, 4000)], acc.at[pl.ds(96000, 4000)])

    zv = jnp.zeros((16,), jnp.float32)
    for i in range(R * 8):
        val2[i // 8, pl.ds((i % 8) * 16, 16)] = zv
    for i in range(6400 // 16):
        zbuf[pl.ds(i * 16, 16)] = zv

    @pl.when((c == 0) & (s == 15))
    def _():
        pltpu.sync_copy(zbuf.at[pl.ds(0, 2400)], acc.at[pl.ds(100000, 2400)])

    @pl.when(c == 1)
    def _():
        pltpu.sync_copy(zbuf, acc.at[pl.ds(s * 6400, 6400)])

    plsc.subcore_barrier()

    for t in range(TPW // R):
        pltpu.sync_copy(tgt_hbm.at[pl.ds(s * TPW + t * R, R), :], idx2)
        for j in range(R):
            pltpu.sync_copy(val2.at[j], acc.at[idx2.at[j]])

    plsc.subcore_barrier()

    def chunk_body(k, carry):
        cid = k * W + wid

        @pl.when(cid < NCHUNK)
        def _():
            base = cid * CH
            pltpu.sync_copy(u_hbm.at[pl.ds(base, CH)], u_v)
            pltpu.sync_copy(wi_hbm.at[pl.ds(base, CH)], wi_v)
            pltpu.sync_copy(v2d_hbm.at[pl.ds(cid * R, R), :], idx2)
            for i in range(CH // 16):
                ui = u_v[pl.ds(i * 16, 16)]
                wi = wi_v[pl.ds(i * 16, 16)]
                xg = plsc.load_gather(x_v, [ui])
                wg = plsc.load_gather(w_v, [wi])
                val2[i // 8, pl.ds((i % 8) * 16, 16)] = xg * wg
            descs = [pltpu.async_copy(val2.at[j], acc.at[idx2.at[j]], sem,
                                      add=True) for j in range(R)]
            for d in descs:
                d.wait()

        return carry

    lax.fori_loop(0, CPW, chunk_body, 0)

    plsc.subcore_barrier()

    pltpu.sync_copy(acc.at[pl.ds(s * 6400, 6400)], zbuf)

    @pl.when(c == 0)
    def _():
        pltpu.sync_copy(zbuf, out0.at[pl.ds(s * 6400, 6400)])

    @pl.when(c == 1)
    def _():
        pltpu.sync_copy(zbuf, out1.at[pl.ds(s * 6400, 6400)])


def _sc_scatter(xf, weights, u, v2d, w_idx, tgt2d):
    mesh = plsc.VectorSubcoreMesh(core_axis_name="c", subcore_axis_name="s",
                                  num_cores=NC, num_subcores=NS)
    return pl.kernel(
        _sc_body,
        out_type=(jax.ShapeDtypeStruct((NPAD,), jnp.float32),
                  jax.ShapeDtypeStruct((NPAD,), jnp.float32)),
        mesh=mesh,
        compiler_params=pltpu.CompilerParams(needs_layout_passes=False),
        scratch_types=[
            pltpu.VMEM((N,), jnp.float32),
            pltpu.VMEM((NWT,), jnp.float32),
            pltpu.VMEM((CH,), jnp.int32),
            pltpu.VMEM((CH,), jnp.int32),
            pltpu.VMEM((R, 128), jnp.int32),
            pltpu.VMEM((R, 128), jnp.float32),
            pltpu.VMEM((6400,), jnp.float32),
            pltpu.VMEM_SHARED((NPAD,), jnp.float32),
            pltpu.SemaphoreType.DMA,
        ],
    )(xf, weights, u, v2d, w_idx, tgt2d)


def _add_body(a_ref, b_ref, o_ref):
    o_ref[...] = a_ref[...] + b_ref[...]


def kernel(x, weights, u, v, w_idx, targets):
    xf = x.reshape(N)
    v2d = v.reshape(E // 128, 128)
    pad = N + jnp.arange(TROWS * 128 - NT, dtype=jnp.int32) % (NPAD - N)
    tgt2d = jnp.concatenate([targets, pad]).reshape(TROWS, 128)
    acc0, acc1 = _sc_scatter(xf, weights, u, v2d, w_idx, tgt2d)
    out2d = pl.pallas_call(
        _add_body,
        out_shape=jax.ShapeDtypeStruct((NPAD // 128, 128), jnp.float32),
    )(acc0.reshape(NPAD // 128, 128), acc1.reshape(NPAD // 128, 128))
    return out2d.reshape(NPAD)[:N].reshape(N, 1)


# speedup vs baseline: 466.2494x; 1.4853x over previous
import jax
import jax.numpy as jnp
from jax import lax
from jax.experimental import pallas as pl
from jax.experimental.pallas import tpu as pltpu
from jax.experimental.pallas import tpu_sc as plsc

N = 100000
E = 6400000
NWT = 1024
NT = 50000
NPAD = 102400

NC = 2
NS = 16
W = NC * NS

CH = 2048
R = CH // 128
NCHUNK = E // CH
CPW = -(-NCHUNK // W)

TROWS = 512
TPW = TROWS // NS

SEG = 6400


def _sc_body(x_hbm, w_hbm, u_hbm, v2d_hbm, wi_hbm, tgt_hbm,
             out0, out1,
             x_v, w_v, u_v0, u_v1, wi_v0, wi_v1, idx0, idx1, val0, val1,
             zbuf, acc, lsem0, lsem1, ssem0, ssem1):
    c = lax.axis_index("c")
    s = lax.axis_index("s")
    wid = c * NS + s
    ubuf = (u_v0, u_v1)
    wibuf = (wi_v0, wi_v1)
    ibuf = (idx0, idx1)
    vbuf = (val0, val1)
    lsem = (lsem0, lsem1)
    ssem = (ssem0, ssem1)

    pltpu.sync_copy(x_hbm, x_v)
    pltpu.sync_copy(w_hbm, w_v)

    zv = jnp.zeros((16,), jnp.float32)
    for i in range(R * 8):
        val0[i // 8, pl.ds((i % 8) * 16, 16)] = zv
    for i in range(2048 // 16):
        zbuf[pl.ds(i * 16, 16)] = zv

    @pl.when(c == 0)
    def _():
        @pl.when(s < 15)
        def _():
            pltpu.sync_copy(x_v.at[pl.ds(s * SEG, SEG)],
                            acc.at[pl.ds(s * SEG, SEG)])

        @pl.when(s == 15)
        def _():
            pltpu.sync_copy(x_v.at[pl.ds(96000, 4000)],
                            acc.at[pl.ds(96000, 4000)])
            pltpu.sync_copy(zbuf, acc.at[pl.ds(100000, 2048)])
            pltpu.sync_copy(zbuf.at[pl.ds(0, 352)], acc.at[pl.ds(102048, 352)])

    @pl.when(c == 1)
    def _():
        for t in range(3):
            pltpu.sync_copy(zbuf, acc.at[pl.ds(s * SEG + t * 2048, 2048)])
        pltpu.sync_copy(zbuf.at[pl.ds(0, 256)], acc.at[pl.ds(s * SEG + 6144, 256)])

    plsc.subcore_barrier()

    for t in range(TPW // R):
        pltpu.sync_copy(tgt_hbm.at[pl.ds(s * TPW + t * R, R), :], idx0)
        for j in range(R):
            pltpu.make_async_copy(val0.at[j], acc.at[idx0.at[j]], ssem0).start()
        for j in range(R):
            pltpu.make_async_copy(val0.at[j], acc.at[idx0.at[j]], ssem0).wait()

    plsc.subcore_barrier()

    def valid(k):
        return (k >= 0) & (k * W + wid < NCHUNK)

    def start_loads(b, k):
        cid = k * W + wid
        base = cid * CH
        pltpu.make_async_copy(u_hbm.at[pl.ds(base, CH)], ubuf[b], lsem[b]).start()
        pltpu.make_async_copy(wi_hbm.at[pl.ds(base, CH)], wibuf[b], lsem[b]).start()
        pltpu.make_async_copy(v2d_hbm.at[pl.ds(cid * R, R), :], ibuf[b], lsem[b]).start()

    def wait_loads(b):
        pltpu.make_async_copy(u_hbm.at[pl.ds(0, CH)], ubuf[b], lsem[b]).wait()
        pltpu.make_async_copy(wi_hbm.at[pl.ds(0, CH)], wibuf[b], lsem[b]).wait()
        pltpu.make_async_copy(v2d_hbm.at[pl.ds(0, R), :], ibuf[b], lsem[b]).wait()

    def compute(b):
        for i in range(CH // 16):
            ui = ubuf[b][pl.ds(i * 16, 16)]
            wi = wibuf[b][pl.ds(i * 16, 16)]
            xg = plsc.load_gather(x_v, [ui])
            wg = plsc.load_gather(w_v, [wi])
            vbuf[b][i // 8, pl.ds((i % 8) * 16, 16)] = xg * wg

    def fire_scatter(b):
        for j in range(R):
            pltpu.make_async_copy(vbuf[b].at[j], acc.at[ibuf[b].at[j]],
                                  ssem[b]).start(add=True)

    def drain_scatter(b):
        for j in range(R):
            pltpu.make_async_copy(vbuf[b].at[j], acc.at[ibuf[b].at[j]],
                                  ssem[b]).wait()

    start_loads(0, jnp.int32(0))

    def pair_body(i, carry):
        k0 = 2 * i
        k1 = 2 * i + 1

        @pl.when(valid(k0))
        def _():
            wait_loads(0)
            compute(0)

        @pl.when(valid(k0 - 1))
        def _():
            drain_scatter(1)

        @pl.when(valid(k1))
        def _():
            start_loads(1, k1)

        @pl.when(valid(k0))
        def _():
            fire_scatter(0)

        @pl.when(valid(k1))
        def _():
            wait_loads(1)
            compute(1)

        @pl.when(valid(k0))
        def _():
            drain_scatter(0)

        @pl.when(valid(k1 + 1))
        def _():
            start_loads(0, k1 + 1)

        @pl.when(valid(k1))
        def _():
            fire_scatter(1)

        return carry

    lax.fori_loop(0, CPW // 2, pair_body, 0)

    @pl.when(valid(jnp.int32(CPW - 1)))
    def _():
        drain_scatter(1)

    plsc.subcore_barrier()

    pltpu.sync_copy(acc.at[pl.ds(s * SEG, SEG)], x_v.at[pl.ds(0, SEG)])

    @pl.when(c == 0)
    def _():
        pltpu.sync_copy(x_v.at[pl.ds(0, SEG)], out0.at[pl.ds(s * SEG, SEG)])

    @pl.when(c == 1)
    def _():
        pltpu.sync_copy(x_v.at[pl.ds(0, SEG)], out1.at[pl.ds(s * SEG, SEG)])


def _sc_scatter(xf, weights, u, v2d, w_idx, tgt2d):
    mesh = plsc.VectorSubcoreMesh(core_axis_name="c", subcore_axis_name="s",
                                  num_cores=NC, num_subcores=NS)
    return pl.kernel(
        _sc_body,
        out_type=(jax.ShapeDtypeStruct((NPAD,), jnp.float32),
                  jax.ShapeDtypeStruct((NPAD,), jnp.float32)),
        mesh=mesh,
        compiler_params=pltpu.CompilerParams(needs_layout_passes=False),
        scratch_types=[
            pltpu.VMEM((N,), jnp.float32),
            pltpu.VMEM((NWT,), jnp.float32),
            pltpu.VMEM((CH,), jnp.int32),
            pltpu.VMEM((CH,), jnp.int32),
            pltpu.VMEM((CH,), jnp.int32),
            pltpu.VMEM((CH,), jnp.int32),
            pltpu.VMEM((R, 128), jnp.int32),
            pltpu.VMEM((R, 128), jnp.int32),
            pltpu.VMEM((R, 128), jnp.float32),
            pltpu.VMEM((R, 128), jnp.float32),
            pltpu.VMEM((2048,), jnp.float32),
            pltpu.VMEM_SHARED((NPAD,), jnp.float32),
            pltpu.SemaphoreType.DMA,
            pltpu.SemaphoreType.DMA,
            pltpu.SemaphoreType.DMA,
            pltpu.SemaphoreType.DMA,
        ],
    )(xf, weights, u, v2d, w_idx, tgt2d)


def _add_body(a_ref, b_ref, o_ref):
    o_ref[...] = a_ref[...] + b_ref[...]


def kernel(x, weights, u, v, w_idx, targets):
    xf = x.reshape(N)
    v2d = v.reshape(E // 128, 128)
    pad = N + jnp.arange(TROWS * 128 - NT, dtype=jnp.int32) % (NPAD - N)
    tgt2d = jnp.concatenate([targets, pad]).reshape(TROWS, 128)
    acc0, acc1 = _sc_scatter(xf, weights, u, v2d, w_idx, tgt2d)
    out2d = pl.pallas_call(
        _add_body,
        out_shape=jax.ShapeDtypeStruct((NPAD // 128, 128), jnp.float32),
    )(acc0.reshape(NPAD // 128, 128), acc1.reshape(NPAD // 128, 128))
    return out2d.reshape(NPAD)[:N].reshape(N, 1)


# speedup vs baseline: 489.7994x; 1.0505x over previous
import jax
import jax.numpy as jnp
from jax import lax
from jax.experimental import pallas as pl
from jax.experimental.pallas import tpu as pltpu
from jax.experimental.pallas import tpu_sc as plsc

N = 100000
E = 6400000
NWT = 1024
NT = 50000
NPAD = 102400

NC = 2
NS = 16
W = NC * NS

CH = 2048
R = CH // 128
NCHUNK = E // CH
CPW = -(-NCHUNK // W)

TROWS = 512
TPW = TROWS // NS

SEG = 6400


def _sc_body(x_hbm, w_hbm, u_hbm, v_hbm, wi_hbm, tgt_hbm,
             out0, out1,
             x_v, w_v, u_v0, u_v1, wi_v0, wi_v1, idx0, idx1, val0, val1,
             zbuf, acc, lsem0, lsem1, ssem0, ssem1):
    c = lax.axis_index("c")
    s = lax.axis_index("s")
    wid = c * NS + s
    ubuf = (u_v0, u_v1)
    wibuf = (wi_v0, wi_v1)
    ibuf = (idx0, idx1)
    vbuf = (val0, val1)
    lsem = (lsem0, lsem1)
    ssem = (ssem0, ssem1)

    pltpu.sync_copy(x_hbm, x_v)
    pltpu.sync_copy(w_hbm, w_v)

    zv = jnp.zeros((16,), jnp.float32)
    for i in range(CH // 16):
        val0[pl.ds(i * 16, 16)] = zv
    for i in range(2048 // 16):
        zbuf[pl.ds(i * 16, 16)] = zv

    @pl.when(c == 0)
    def _():
        @pl.when(s < 15)
        def _():
            pltpu.sync_copy(x_v.at[pl.ds(s * SEG, SEG)],
                            acc.at[pl.ds(s * SEG, SEG)])

        @pl.when(s == 15)
        def _():
            pltpu.sync_copy(x_v.at[pl.ds(96000, 4000)],
                            acc.at[pl.ds(96000, 4000)])
            pltpu.sync_copy(zbuf, acc.at[pl.ds(100000, 2048)])
            pltpu.sync_copy(zbuf.at[pl.ds(0, 352)], acc.at[pl.ds(102048, 352)])

    @pl.when(c == 1)
    def _():
        for t in range(3):
            pltpu.sync_copy(zbuf, acc.at[pl.ds(s * SEG + t * 2048, 2048)])
        pltpu.sync_copy(zbuf.at[pl.ds(0, 256)], acc.at[pl.ds(s * SEG + 6144, 256)])

    plsc.subcore_barrier()

    for t in range(TPW * 128 // CH):
        pltpu.sync_copy(tgt_hbm.at[pl.ds(s * TPW * 128 + t * CH, CH)], idx0)
        pltpu.make_async_copy(val0, acc.at[idx0], ssem0).start()
        pltpu.make_async_copy(val0, acc.at[idx0], ssem0).wait()

    plsc.subcore_barrier()

    def valid(k):
        return (k >= 0) & (k * W + wid < NCHUNK)

    def start_loads(b, k):
        cid = k * W + wid
        base = cid * CH
        pltpu.make_async_copy(u_hbm.at[pl.ds(base, CH)], ubuf[b], lsem[b]).start()
        pltpu.make_async_copy(wi_hbm.at[pl.ds(base, CH)], wibuf[b], lsem[b]).start()
        pltpu.make_async_copy(v_hbm.at[pl.ds(base, CH)], ibuf[b], lsem[b]).start()

    def wait_loads(b):
        pltpu.make_async_copy(u_hbm.at[pl.ds(0, CH)], ubuf[b], lsem[b]).wait()
        pltpu.make_async_copy(wi_hbm.at[pl.ds(0, CH)], wibuf[b], lsem[b]).wait()
        pltpu.make_async_copy(v_hbm.at[pl.ds(0, CH)], ibuf[b], lsem[b]).wait()

    def compute(b):
        for i in range(CH // 16):
            ui = ubuf[b][pl.ds(i * 16, 16)]
            wi = wibuf[b][pl.ds(i * 16, 16)]
            xg = plsc.load_gather(x_v, [ui])
            wg = plsc.load_gather(w_v, [wi])
            vbuf[b][pl.ds(i * 16, 16)] = xg * wg

    def fire_scatter(b):
        pltpu.make_async_copy(vbuf[b], acc.at[ibuf[b]], ssem[b]).start(add=True)

    def drain_scatter(b):
        pltpu.make_async_copy(vbuf[b], acc.at[ibuf[b]], ssem[b]).wait()

    start_loads(0, jnp.int32(0))

    def pair_body(i, carry):
        k0 = 2 * i
        k1 = 2 * i + 1

        @pl.when(valid(k0))
        def _():
            wait_loads(0)
            compute(0)

        @pl.when(valid(k0 - 1))
        def _():
            drain_scatter(1)

        @pl.when(valid(k1))
        def _():
            start_loads(1, k1)

        @pl.when(valid(k0))
        def _():
            fire_scatter(0)

        @pl.when(valid(k1))
        def _():
            wait_loads(1)
            compute(1)

        @pl.when(valid(k0))
        def _():
            drain_scatter(0)

        @pl.when(valid(k1 + 1))
        def _():
            start_loads(0, k1 + 1)

        @pl.when(valid(k1))
        def _():
            fire_scatter(1)

        return carry

    lax.fori_loop(0, CPW // 2, pair_body, 0)

    @pl.when(valid(jnp.int32(CPW - 1)))
    def _():
        drain_scatter(1)

    plsc.subcore_barrier()

    pltpu.sync_copy(acc.at[pl.ds(s * SEG, SEG)], x_v.at[pl.ds(0, SEG)])

    @pl.when(c == 0)
    def _():
        pltpu.sync_copy(x_v.at[pl.ds(0, SEG)], out0.at[pl.ds(s * SEG, SEG)])

    @pl.when(c == 1)
    def _():
        pltpu.sync_copy(x_v.at[pl.ds(0, SEG)], out1.at[pl.ds(s * SEG, SEG)])


def _sc_scatter(xf, weights, u, v, w_idx, tgt1d):
    mesh = plsc.VectorSubcoreMesh(core_axis_name="c", subcore_axis_name="s",
                                  num_cores=NC, num_subcores=NS)
    return pl.kernel(
        _sc_body,
        out_type=(jax.ShapeDtypeStruct((NPAD,), jnp.float32),
                  jax.ShapeDtypeStruct((NPAD,), jnp.float32)),
        mesh=mesh,
        compiler_params=pltpu.CompilerParams(needs_layout_passes=False),
        scratch_types=[
            pltpu.VMEM((N,), jnp.float32),
            pltpu.VMEM((NWT,), jnp.float32),
            pltpu.VMEM((CH,), jnp.int32),
            pltpu.VMEM((CH,), jnp.int32),
            pltpu.VMEM((CH,), jnp.int32),
            pltpu.VMEM((CH,), jnp.int32),
            pltpu.VMEM((CH,), jnp.int32),
            pltpu.VMEM((CH,), jnp.int32),
            pltpu.VMEM((CH,), jnp.float32),
            pltpu.VMEM((CH,), jnp.float32),
            pltpu.VMEM((2048,), jnp.float32),
            pltpu.VMEM_SHARED((NPAD,), jnp.float32),
            pltpu.SemaphoreType.DMA,
            pltpu.SemaphoreType.DMA,
            pltpu.SemaphoreType.DMA,
            pltpu.SemaphoreType.DMA,
        ],
    )(xf, weights, u, v, w_idx, tgt1d)


def _add_body(a_ref, b_ref, o_ref):
    o_ref[...] = a_ref[...] + b_ref[...]


def kernel(x, weights, u, v, w_idx, targets):
    xf = x.reshape(N)
    pad = N + jnp.arange(TROWS * 128 - NT, dtype=jnp.int32) % (NPAD - N)
    tgt1d = jnp.concatenate([targets, pad])
    acc0, acc1 = _sc_scatter(xf, weights, u, v, w_idx, tgt1d)
    out2d = pl.pallas_call(
        _add_body,
        out_shape=jax.ShapeDtypeStruct((NPAD // 128, 128), jnp.float32),
    )(acc0.reshape(NPAD // 128, 128), acc1.reshape(NPAD // 128, 128))
    return out2d.reshape(NPAD)[:N].reshape(N, 1)


# speedup vs baseline: 735.9974x; 1.5027x over previous
import jax
import jax.numpy as jnp
from jax import lax
from jax.experimental import pallas as pl
from jax.experimental.pallas import tpu as pltpu
from jax.experimental.pallas import tpu_sc as plsc

N = 100000
E = 6400000
NWT = 1024
NT = 50000
NPAD = 102400

NC = 2
NS = 16
W = NC * NS

CH = 2048
R = CH // 128
NCHUNK = E // CH
CPW = -(-NCHUNK // W)

TROWS = 512
TPW = TROWS // NS

SEG = 6400


def _sc_body(x_hbm, w_hbm, u_hbm, v_hbm, wi_hbm, tgt_hbm,
             out0, out1,
             x_v, w_v, u_v0, u_v1, wi_v0, wi_v1, idx0, idx1, val0, val1,
             zbuf, acc, lsem0, lsem1, ssem0, ssem1):
    c = lax.axis_index("c")
    s = lax.axis_index("s")
    wid = c * NS + s
    ubuf = (u_v0, u_v1)
    wibuf = (wi_v0, wi_v1)
    ibuf = (idx0, idx1)
    vbuf = (val0, val1)
    lsem = (lsem0, lsem1)
    ssem = (ssem0, ssem1)

    pltpu.sync_copy(x_hbm, x_v)
    pltpu.sync_copy(w_hbm, w_v)

    zv = jnp.zeros((16,), jnp.float32)
    for i in range(CH // 16):
        val0[pl.ds(i * 16, 16)] = zv
    for i in range(2048 // 16):
        zbuf[pl.ds(i * 16, 16)] = zv

    @pl.when(c == 0)
    def _():
        @pl.when(s < 15)
        def _():
            pltpu.sync_copy(x_v.at[pl.ds(s * SEG, SEG)],
                            acc.at[pl.ds(s * SEG, SEG)])

        @pl.when(s == 15)
        def _():
            pltpu.sync_copy(x_v.at[pl.ds(96000, 4000)],
                            acc.at[pl.ds(96000, 4000)])
            pltpu.sync_copy(zbuf, acc.at[pl.ds(100000, 2048)])
            pltpu.sync_copy(zbuf.at[pl.ds(0, 352)], acc.at[pl.ds(102048, 352)])

    @pl.when(c == 1)
    def _():
        for t in range(3):
            pltpu.sync_copy(zbuf, acc.at[pl.ds(s * SEG + t * 2048, 2048)])
        pltpu.sync_copy(zbuf.at[pl.ds(0, 256)], acc.at[pl.ds(s * SEG + 6144, 256)])

    plsc.subcore_barrier()

    for t in range(TPW * 128 // CH):
        pltpu.sync_copy(tgt_hbm.at[pl.ds(s * TPW * 128 + t * CH, CH)], idx0)
        pltpu.make_async_copy(val0, acc.at[idx0], ssem0).start()
        pltpu.make_async_copy(val0, acc.at[idx0], ssem0).wait()

    plsc.subcore_barrier()

    def valid(k):
        return (k >= 0) & (k * W + wid < NCHUNK)

    def start_loads(b, k):
        cid = k * W + wid
        base = cid * CH
        pltpu.make_async_copy(u_hbm.at[pl.ds(base, CH)], ubuf[b], lsem[b]).start()
        pltpu.make_async_copy(wi_hbm.at[pl.ds(base, CH)], wibuf[b], lsem[b]).start()
        pltpu.make_async_copy(v_hbm.at[pl.ds(base, CH)], ibuf[b], lsem[b]).start()

    def wait_loads(b):
        pltpu.make_async_copy(u_hbm.at[pl.ds(0, CH)], ubuf[b], lsem[b]).wait()
        pltpu.make_async_copy(wi_hbm.at[pl.ds(0, CH)], wibuf[b], lsem[b]).wait()
        pltpu.make_async_copy(v_hbm.at[pl.ds(0, CH)], ibuf[b], lsem[b]).wait()

    def compute(b):
        @plsc.parallel_loop(0, CH, 16, unroll=4)
        def _(i):
            ui = ubuf[b][pl.ds(i, 16)]
            wi = wibuf[b][pl.ds(i, 16)]
            xg = plsc.load_gather(x_v, [ui])
            wg = plsc.load_gather(w_v, [wi])
            vbuf[b][pl.ds(i, 16)] = xg * wg

    def fire_scatter(b):
        pltpu.make_async_copy(vbuf[b], acc.at[ibuf[b]], ssem[b]).start(add=True)

    def drain_scatter(b):
        pltpu.make_async_copy(vbuf[b], acc.at[ibuf[b]], ssem[b]).wait()

    start_loads(0, jnp.int32(0))

    def pair_body(i, carry):
        k0 = 2 * i
        k1 = 2 * i + 1

        @pl.when(valid(k0))
        def _():
            wait_loads(0)
            compute(0)

        @pl.when(valid(k0 - 1))
        def _():
            drain_scatter(1)

        @pl.when(valid(k1))
        def _():
            start_loads(1, k1)

        @pl.when(valid(k0))
        def _():
            fire_scatter(0)

        @pl.when(valid(k1))
        def _():
            wait_loads(1)
            compute(1)

        @pl.when(valid(k0))
        def _():
            drain_scatter(0)

        @pl.when(valid(k1 + 1))
        def _():
            start_loads(0, k1 + 1)

        @pl.when(valid(k1))
        def _():
            fire_scatter(1)

        return carry

    lax.fori_loop(0, CPW // 2, pair_body, 0)

    @pl.when(valid(jnp.int32(CPW - 1)))
    def _():
        drain_scatter(1)

    plsc.subcore_barrier()

    pltpu.sync_copy(acc.at[pl.ds(s * SEG, SEG)], x_v.at[pl.ds(0, SEG)])

    @pl.when(c == 0)
    def _():
        pltpu.sync_copy(x_v.at[pl.ds(0, SEG)], out0.at[pl.ds(s * SEG, SEG)])

    @pl.when(c == 1)
    def _():
        pltpu.sync_copy(x_v.at[pl.ds(0, SEG)], out1.at[pl.ds(s * SEG, SEG)])


def _sc_scatter(xf, weights, u, v, w_idx, tgt1d):
    mesh = plsc.VectorSubcoreMesh(core_axis_name="c", subcore_axis_name="s",
                                  num_cores=NC, num_subcores=NS)
    return pl.kernel(
        _sc_body,
        out_type=(jax.ShapeDtypeStruct((NPAD,), jnp.float32),
                  jax.ShapeDtypeStruct((NPAD,), jnp.float32)),
        mesh=mesh,
        compiler_params=pltpu.CompilerParams(needs_layout_passes=False),
        scratch_types=[
            pltpu.VMEM((N,), jnp.float32),
            pltpu.VMEM((NWT,), jnp.float32),
            pltpu.VMEM((CH,), jnp.int32),
            pltpu.VMEM((CH,), jnp.int32),
            pltpu.VMEM((CH,), jnp.int32),
            pltpu.VMEM((CH,), jnp.int32),
            pltpu.VMEM((CH,), jnp.int32),
            pltpu.VMEM((CH,), jnp.int32),
            pltpu.VMEM((CH,), jnp.float32),
            pltpu.VMEM((CH,), jnp.float32),
            pltpu.VMEM((2048,), jnp.float32),
            pltpu.VMEM_SHARED((NPAD,), jnp.float32),
            pltpu.SemaphoreType.DMA,
            pltpu.SemaphoreType.DMA,
            pltpu.SemaphoreType.DMA,
            pltpu.SemaphoreType.DMA,
        ],
    )(xf, weights, u, v, w_idx, tgt1d)


def _add_body(a_ref, b_ref, o_ref):
    o_ref[...] = a_ref[...] + b_ref[...]


def kernel(x, weights, u, v, w_idx, targets):
    xf = x.reshape(N)
    pad = N + jnp.arange(TROWS * 128 - NT, dtype=jnp.int32) % (NPAD - N)
    tgt1d = jnp.concatenate([targets, pad])
    acc0, acc1 = _sc_scatter(xf, weights, u, v, w_idx, tgt1d)
    out2d = pl.pallas_call(
        _add_body,
        out_shape=jax.ShapeDtypeStruct((NPAD // 128, 128), jnp.float32),
    )(acc0.reshape(NPAD // 128, 128), acc1.reshape(NPAD // 128, 128))
    return out2d.reshape(NPAD)[:N].reshape(N, 1)


# speedup vs baseline: 1031.5928x; 1.4016x over previous
import jax
import jax.numpy as jnp
from jax import lax
from jax.experimental import pallas as pl
from jax.experimental.pallas import tpu as pltpu
from jax.experimental.pallas import tpu_sc as plsc

N = 100000
E = 6400000
NWT = 1024
NT = 50000
NPAD = 102400

NC = 2
NS = 16
W = NC * NS

CH = 2560
NCHUNK = E // CH
CPW = -(-NCHUNK // W)
KTOT = (CPW + 2 + 5) // 6

TT = 32 * CH
TPT = TT // NS

SEG = NPAD // NS


def _sc_body(x_hbm, w_hbm, u_hbm, v_hbm, wi_hbm, tgt_hbm,
             out0, out1,
             x_v, w_v, u_v0, u_v1, wi_v0, wi_v1, idx0, idx1, idx2, val0,
             val1, lsem0, lsem1, ssem0, ssem1, ssem2, acc):
    c = lax.axis_index("c")
    s = lax.axis_index("s")
    wid = c * NS + s
    ubuf = (u_v0, u_v1)
    wibuf = (wi_v0, wi_v1)
    ibuf = (idx0, idx1, idx2)
    vbuf = (val0, val1)
    lsem = (lsem0, lsem1)
    ssem = (ssem0, ssem1, ssem2)

    def start_loads(bu, bi, k):
        base = (k * W + wid) * CH
        pltpu.make_async_copy(u_hbm.at[pl.ds(base, CH)], ubuf[bu], lsem[bu]).start()
        pltpu.make_async_copy(wi_hbm.at[pl.ds(base, CH)], wibuf[bu], lsem[bu]).start()
        pltpu.make_async_copy(v_hbm.at[pl.ds(base, CH)], ibuf[bi], lsem[bu]).start()

    def wait_loads(bu, bi):
        pltpu.make_async_copy(u_hbm.at[pl.ds(0, CH)], ubuf[bu], lsem[bu]).wait()
        pltpu.make_async_copy(wi_hbm.at[pl.ds(0, CH)], wibuf[bu], lsem[bu]).wait()
        pltpu.make_async_copy(v_hbm.at[pl.ds(0, CH)], ibuf[bi], lsem[bu]).wait()

    def compute(bu, bv):
        @plsc.parallel_loop(0, CH, 16, unroll=4)
        def _(i):
            ui = ubuf[bu][pl.ds(i, 16)]
            wi = wibuf[bu][pl.ds(i, 16)]
            xg = plsc.load_gather(x_v, [ui])
            wg = plsc.load_gather(w_v, [wi])
            vbuf[bv][pl.ds(i, 16)] = xg * wg

    def fire_scatter(bv, bi):
        pltpu.make_async_copy(vbuf[bv], acc.at[ibuf[bi]], ssem[bi]).start(add=True)

    def drain_scatter(bv, bi):
        pltpu.make_async_copy(vbuf[bv], acc.at[ibuf[bi]], ssem[bi]).wait()

    pltpu.make_async_copy(x_hbm, x_v, ssem1).start()
    pltpu.make_async_copy(w_hbm, w_v, ssem1).start()
    start_loads(0, 0, jnp.int32(0))

    zv = jnp.zeros((16,), jnp.float32)

    @plsc.parallel_loop(0, CH, 16)
    def _(i):
        val0[pl.ds(i, 16)] = zv

    pltpu.make_async_copy(x_hbm, x_v, ssem1).wait()
    pltpu.make_async_copy(w_hbm, w_v, ssem1).wait()

    @pl.when(c == 0)
    def _():
        @pl.when(s < 15)
        def _():
            pltpu.sync_copy(x_v.at[pl.ds(s * SEG, SEG)],
                            acc.at[pl.ds(s * SEG, SEG)])

        @pl.when(s == 15)
        def _():
            pltpu.sync_copy(x_v.at[pl.ds(96000, 4000)],
                            acc.at[pl.ds(96000, 4000)])
            pltpu.sync_copy(val0.at[pl.ds(0, 2400)], acc.at[pl.ds(100000, 2400)])

    @pl.when(c == 1)
    def _():
        pltpu.sync_copy(val0, acc.at[pl.ds(s * SEG, CH)])
        pltpu.sync_copy(val0, acc.at[pl.ds(s * SEG + CH, CH)])
        pltpu.sync_copy(val0.at[pl.ds(0, 1280)], acc.at[pl.ds(s * SEG + 2 * CH, 1280)])

    plsc.subcore_barrier()

    for t in range(TPT // CH):
        pltpu.sync_copy(tgt_hbm.at[pl.ds(s * TPT + t * CH, CH)], idx2)
        pltpu.make_async_copy(val0, acc.at[idx2], ssem0).start()
        pltpu.make_async_copy(val0, acc.at[idx2], ssem0).wait()

    plsc.subcore_barrier()

    def valid(k):
        return (k >= 0) & (k * W + wid < NCHUNK)

    def six_body(it, carry):
        for j in range(6):
            k = 6 * it + j

            @pl.when(valid(k))
            def _():
                wait_loads(j % 2, j % 3)

            @pl.when(valid(k - 2))
            def _():
                drain_scatter(j % 2, (j + 1) % 3)

            @pl.when(valid(k + 1))
            def _():
                start_loads((j + 1) % 2, (j + 1) % 3, k + 1)

            @pl.when(valid(k))
            def _():
                compute(j % 2, j % 2)
                fire_scatter(j % 2, j % 3)

        return carry

    lax.fori_loop(0, KTOT, six_body, 0)

    plsc.subcore_barrier()

    pltpu.sync_copy(acc.at[pl.ds(s * SEG, SEG)], x_v.at[pl.ds(0, SEG)])

    @pl.when(c == 0)
    def _():
        pltpu.sync_copy(x_v.at[pl.ds(0, SEG)], out0.at[pl.ds(s * SEG, SEG)])

    @pl.when(c == 1)
    def _():
        pltpu.sync_copy(x_v.at[pl.ds(0, SEG)], out1.at[pl.ds(s * SEG, SEG)])


def _sc_scatter(xf, weights, u, v, w_idx, tgt1d):
    mesh = plsc.VectorSubcoreMesh(core_axis_name="c", subcore_axis_name="s",
                                  num_cores=NC, num_subcores=NS)
    return pl.kernel(
        _sc_body,
        out_type=(jax.ShapeDtypeStruct((NPAD,), jnp.float32),
                  jax.ShapeDtypeStruct((NPAD,), jnp.float32)),
        mesh=mesh,
        compiler_params=pltpu.CompilerParams(needs_layout_passes=False),
        scratch_types=[
            pltpu.VMEM((N,), jnp.float32),
            pltpu.VMEM((NWT,), jnp.float32),
            pltpu.VMEM((CH,), jnp.int32),
            pltpu.VMEM((CH,), jnp.int32),
            pltpu.VMEM((CH,), jnp.int32),
            pltpu.VMEM((CH,), jnp.int32),
            pltpu.VMEM((CH,), jnp.int32),
            pltpu.VMEM((CH,), jnp.int32),
            pltpu.VMEM((CH,), jnp.int32),
            pltpu.VMEM((CH,), jnp.float32),
            pltpu.VMEM((CH,), jnp.float32),
            pltpu.SemaphoreType.DMA,
            pltpu.SemaphoreType.DMA,
            pltpu.SemaphoreType.DMA,
            pltpu.SemaphoreType.DMA,
            pltpu.SemaphoreType.DMA,
            pltpu.VMEM_SHARED((NPAD,), jnp.float32),
        ],
    )(xf, weights, u, v, w_idx, tgt1d)


def _add_body(a_ref, b_ref, o_ref):
    o_ref[...] = a_ref[...] + b_ref[...]


def kernel(x, weights, u, v, w_idx, targets):
    xf = x.reshape(N)
    pad = N + jnp.arange(TT - NT, dtype=jnp.int32) % (NPAD - N)
    tgt1d = jnp.concatenate([targets, pad])
    acc0, acc1 = _sc_scatter(xf, weights, u, v, w_idx, tgt1d)
    out2d = pl.pallas_call(
        _add_body,
        out_shape=jax.ShapeDtypeStruct((NPAD // 128, 128), jnp.float32),
    )(acc0.reshape(NPAD // 128, 128), acc1.reshape(NPAD // 128, 128))
    return out2d.reshape(NPAD)[:N].reshape(N, 1)
